# baseline (speedup 1.0000x reference)
import jax
import jax.numpy as jnp
from jax.experimental import pallas as pl
from jax.experimental.pallas import tpu as pltpu


def _fused_mlp_kernel(t0_ref, t1_ref, x_ref,
                      gva_ref, gvb_ref, uva_ref, uvb_ref,
                      gla_ref, glb_ref, ula_ref, ulb_ref,
                      dva_ref, dvb_ref, dla_ref, dlb_ref,
                      out_ref):
    i = pl.program_id(0)
    maskf = ((t0_ref[:] == 1) & (t1_ref[:] == 1)).astype(jnp.float32)

    bf16, f32 = jnp.bfloat16, jnp.float32
    Dh = gva_ref.shape[0]
    xa = x_ref[:, :Dh].astype(bf16)
    xb = x_ref[:, Dh:].astype(bf16)

    def proj2(a_ref, b_ref):
        return (jnp.dot(xa, a_ref[:].astype(bf16), preferred_element_type=f32)
                + jnp.dot(xb, b_ref[:].astype(bf16), preferred_element_type=f32))

    hv = jax.nn.silu(proj2(gva_ref, gvb_ref)) * proj2(uva_ref, uvb_ref)
    hl = jax.nn.silu(proj2(gla_ref, glb_ref)) * proj2(ula_ref, ulb_ref)

    hv = (hv * maskf).astype(bf16)
    hl = (hl * (1.0 - maskf)).astype(bf16)
    Ch = dva_ref.shape[0]
    contrib = (
        jnp.dot(hv[:, :Ch], dva_ref[:].astype(bf16), preferred_element_type=f32)
        + jnp.dot(hv[:, Ch:], dvb_ref[:].astype(bf16), preferred_element_type=f32)
        + jnp.dot(hl[:, :Ch], dla_ref[:].astype(bf16), preferred_element_type=f32)
        + jnp.dot(hl[:, Ch:], dlb_ref[:].astype(bf16), preferred_element_type=f32)
    )

    @pl.when(i == 0)
    def _():
        out_ref[:] = contrib

    @pl.when(i > 0)
    def _():
        out_ref[:] += contrib


def kernel(hidden_states, token_type_ids, gate_v, up_v, down_v,
           gate_l, up_l, down_l):
    B, L, D = hidden_states.shape
    I = gate_v.shape[1]
    N = B * L
    C = 128
    steps = I // C
    assert steps * C == I
    Dh = D // 2
    Ch = C // 2

    x = hidden_states.reshape(N, D)
    t0 = token_type_ids.reshape(N, 1)
    t_next = jnp.concatenate(
        [token_type_ids[:, 1:],
         jnp.full((B, 1), -1, dtype=token_type_ids.dtype)], axis=1)
    t1 = t_next.reshape(N, 1)

    gu_a = pl.BlockSpec((Dh, C), lambda i: (0, i))
    gu_b = pl.BlockSpec((Dh, C), lambda i: (1, i))
    dn_a = pl.BlockSpec((Ch, D), lambda i: (2 * i, 0))
    dn_b = pl.BlockSpec((Ch, D), lambda i: (2 * i + 1, 0))

    out = pl.pallas_call(
        _fused_mlp_kernel,
        grid=(steps,),
        in_specs=[
            pl.BlockSpec((N, 1), lambda i: (0, 0)),
            pl.BlockSpec((N, 1), lambda i: (0, 0)),
            pl.BlockSpec((N, D), lambda i: (0, 0)),
            gu_a, gu_b,
            gu_a, gu_b,
            gu_a, gu_b,
            gu_a, gu_b,
            dn_a, dn_b,
            dn_a, dn_b,
        ],
        out_specs=pl.BlockSpec((N, D), lambda i: (0, 0)),
        out_shape=jax.ShapeDtypeStruct((N, D), jnp.float32),
        compiler_params=pltpu.CompilerParams(
            dimension_semantics=("arbitrary",),
        ),
    )(t0, t1, x,
      gate_v, gate_v, up_v, up_v, gate_l, gate_l, up_l, up_l,
      down_v, down_v, down_l, down_l)

    return out.reshape(B, L, D)


# speedup vs baseline: 1.0312x; 1.0312x over previous
import jax
import jax.numpy as jnp
from jax.experimental import pallas as pl
from jax.experimental.pallas import tpu as pltpu


def _fused_mlp_kernel(t0_ref, t1_ref, x_ref, gv_ref, uv_ref, gl_ref,
                      ul_ref, dv_ref, dl_ref, out_ref, acc_ref):
    i = pl.program_id(0)
    maskf = ((t0_ref[:] == 1) & (t1_ref[:] == 1)).astype(jnp.float32)

    f32 = jnp.float32
    x = x_ref[:]

    hv = jax.nn.silu(
        jnp.dot(x, gv_ref[:], preferred_element_type=f32)
    ) * jnp.dot(x, uv_ref[:], preferred_element_type=f32)
    hl = jax.nn.silu(
        jnp.dot(x, gl_ref[:], preferred_element_type=f32)
    ) * jnp.dot(x, ul_ref[:], preferred_element_type=f32)

    hv = hv * maskf
    hl = hl * (1.0 - maskf)
    contrib = (
        jnp.dot(hv, dv_ref[:], preferred_element_type=f32)
        + jnp.dot(hl, dl_ref[:], preferred_element_type=f32)
    )

    @pl.when(i == 0)
    def _():
        acc_ref[:] = contrib

    @pl.when(i > 0)
    def _():
        acc_ref[:] += contrib

    @pl.when(i == pl.num_programs(0) - 1)
    def _():
        out_ref[:] = acc_ref[:]


def kernel(hidden_states, token_type_ids, gate_v, up_v, down_v,
           gate_l, up_l, down_l):
    B, L, D = hidden_states.shape
    I = gate_v.shape[1]
    N = B * L
    C = 256
    steps = I // C

    x = hidden_states.reshape(N, D)
    t0 = token_type_ids.reshape(N, 1)
    t_next = jnp.concatenate(
        [token_type_ids[:, 1:],
         jnp.full((B, 1), -1, dtype=token_type_ids.dtype)], axis=1)
    t1 = t_next.reshape(N, 1)

    out = pl.pallas_call(
        _fused_mlp_kernel,
        grid=(steps,),
        in_specs=[
            pl.BlockSpec((N, 1), lambda i: (0, 0)),
            pl.BlockSpec((N, 1), lambda i: (0, 0)),
            pl.BlockSpec((N, D), lambda i: (0, 0)),
            pl.BlockSpec((D, C), lambda i: (0, i)),
            pl.BlockSpec((D, C), lambda i: (0, i)),
            pl.BlockSpec((D, C), lambda i: (0, i)),
            pl.BlockSpec((D, C), lambda i: (0, i)),
            pl.BlockSpec((C, D), lambda i: (i, 0)),
            pl.BlockSpec((C, D), lambda i: (i, 0)),
        ],
        out_specs=pl.BlockSpec((N, D), lambda i: (0, 0)),
        out_shape=jax.ShapeDtypeStruct((N, D), jnp.float32),
        scratch_shapes=[pltpu.VMEM((N, D), jnp.float32)],
        compiler_params=pltpu.CompilerParams(
            dimension_semantics=("arbitrary",),
        ),
    )(t0, t1, x, gate_v, up_v, gate_l, up_l, down_v, down_l)

    return out.reshape(B, L, D)


# speedup vs baseline: 1.0339x; 1.0026x over previous
import jax
import jax.numpy as jnp
from jax.experimental import pallas as pl
from jax.experimental.pallas import tpu as pltpu


def _fused_mlp_kernel(t0_ref, t1_ref, x_ref, gv_ref, uv_ref, gl_ref,
                      ul_ref, dv_ref, dl_ref, out_ref,
                      acc_ref, hv_ref, hl_ref):
    i = pl.program_id(0)
    steps = pl.num_programs(0) - 1
    f32 = jnp.float32

    @pl.when(i > 0)
    def _():
        contrib = (
            jnp.dot(hv_ref[:], dv_ref[:], preferred_element_type=f32)
            + jnp.dot(hl_ref[:], dl_ref[:], preferred_element_type=f32)
        )

        @pl.when(i == 1)
        def _():
            acc_ref[:] = contrib

        @pl.when(i > 1)
        def _():
            acc_ref[:] += contrib

    @pl.when(i < steps)
    def _():
        maskf = ((t0_ref[:] == 1) & (t1_ref[:] == 1)).astype(f32)
        x = x_ref[:]
        hv = jax.nn.silu(
            jnp.dot(x, gv_ref[:], preferred_element_type=f32)
        ) * jnp.dot(x, uv_ref[:], preferred_element_type=f32)
        hl = jax.nn.silu(
            jnp.dot(x, gl_ref[:], preferred_element_type=f32)
        ) * jnp.dot(x, ul_ref[:], preferred_element_type=f32)
        hv_ref[:] = hv * maskf
        hl_ref[:] = hl * (1.0 - maskf)

    @pl.when(i == steps)
    def _():
        out_ref[:] = acc_ref[:]


def kernel(hidden_states, token_type_ids, gate_v, up_v, down_v,
           gate_l, up_l, down_l):
    B, L, D = hidden_states.shape
    I = gate_v.shape[1]
    N = B * L
    C = 256
    steps = I // C

    x = hidden_states.reshape(N, D)
    t0 = token_type_ids.reshape(N, 1)
    t_next = jnp.concatenate(
        [token_type_ids[:, 1:],
         jnp.full((B, 1), -1, dtype=token_type_ids.dtype)], axis=1)
    t1 = t_next.reshape(N, 1)

    last = steps - 1
    out = pl.pallas_call(
        _fused_mlp_kernel,
        grid=(steps + 1,),
        in_specs=[
            pl.BlockSpec((N, 1), lambda i: (0, 0)),
            pl.BlockSpec((N, 1), lambda i: (0, 0)),
            pl.BlockSpec((N, D), lambda i: (0, 0)),
            pl.BlockSpec((D, C), lambda i: (0, jnp.minimum(i, last))),
            pl.BlockSpec((D, C), lambda i: (0, jnp.minimum(i, last))),
            pl.BlockSpec((D, C), lambda i: (0, jnp.minimum(i, last))),
            pl.BlockSpec((D, C), lambda i: (0, jnp.minimum(i, last))),
            pl.BlockSpec((C, D), lambda i: (jnp.maximum(i - 1, 0), 0)),
            pl.BlockSpec((C, D), lambda i: (jnp.maximum(i - 1, 0), 0)),
        ],
        out_specs=pl.BlockSpec((N, D), lambda i: (0, 0)),
        out_shape=jax.ShapeDtypeStruct((N, D), jnp.float32),
        scratch_shapes=[
            pltpu.VMEM((N, D), jnp.float32),
            pltpu.VMEM((N, C), jnp.float32),
            pltpu.VMEM((N, C), jnp.float32),
        ],
        compiler_params=pltpu.CompilerParams(
            dimension_semantics=("arbitrary",),
        ),
    )(t0, t1, x, gate_v, up_v, gate_l, up_l, down_v, down_l)

    return out.reshape(B, L, D)


# speedup vs baseline: 1.0347x; 1.0008x over previous
import jax
import jax.numpy as jnp
from jax.experimental import pallas as pl
from jax.experimental.pallas import tpu as pltpu


def _fused_mlp_kernel(t0_ref, t1_ref, x_ref, gv_ref, uv_ref, gl_ref,
                      ul_ref, dv_ref, dl_ref, out_ref, acc_ref):
    i = pl.program_id(0)
    maskf = ((t0_ref[:] == 1) & (t1_ref[:] == 1)).astype(jnp.float32)

    f32 = jnp.float32
    x = x_ref[:]

    hv = jax.nn.silu(
        jnp.dot(x, gv_ref[:], preferred_element_type=f32)
    ) * jnp.dot(x, uv_ref[:], preferred_element_type=f32)
    hl = jax.nn.silu(
        jnp.dot(x, gl_ref[:], preferred_element_type=f32)
    ) * jnp.dot(x, ul_ref[:], preferred_element_type=f32)

    hv = hv * maskf
    hl = hl * (1.0 - maskf)
    contrib = (
        jnp.dot(hv, dv_ref[:], preferred_element_type=f32)
        + jnp.dot(hl, dl_ref[:], preferred_element_type=f32)
    )

    @pl.when(i == 0)
    def _():
        acc_ref[:] = contrib

    @pl.when(i > 0)
    def _():
        acc_ref[:] += contrib

    @pl.when(i == pl.num_programs(0) - 1)
    def _():
        out_ref[:] = acc_ref[:]


def kernel(hidden_states, token_type_ids, gate_v, up_v, down_v,
           gate_l, up_l, down_l):
    B, L, D = hidden_states.shape
    I = gate_v.shape[1]
    N = B * L
    C = 256
    steps = I // C

    x = hidden_states.reshape(N, D)
    t0 = token_type_ids.reshape(N, 1)
    t_next = jnp.concatenate(
        [token_type_ids[:, 1:],
         jnp.full((B, 1), -1, dtype=token_type_ids.dtype)], axis=1)
    t1 = t_next.reshape(N, 1)

    out = pl.pallas_call(
        _fused_mlp_kernel,
        grid=(steps,),
        in_specs=[
            pl.BlockSpec((N, 1), lambda i: (0, 0)),
            pl.BlockSpec((N, 1), lambda i: (0, 0)),
            pl.BlockSpec((N, D), lambda i: (0, 0)),
            pl.BlockSpec((D, C), lambda i: (0, i)),
            pl.BlockSpec((D, C), lambda i: (0, i)),
            pl.BlockSpec((D, C), lambda i: (0, i)),
            pl.BlockSpec((D, C), lambda i: (0, i)),
            pl.BlockSpec((C, D), lambda i: (i, 0)),
            pl.BlockSpec((C, D), lambda i: (i, 0)),
        ],
        out_specs=pl.BlockSpec((N, D), lambda i: (0, 0)),
        out_shape=jax.ShapeDtypeStruct((N, D), jnp.float32),
        scratch_shapes=[pltpu.VMEM((N, D), jnp.float32)],
        compiler_params=pltpu.CompilerParams(
            dimension_semantics=("arbitrary",),
        ),
    )(t0, t1, x, gate_v, up_v, gate_l, up_l, down_v, down_l)

    return out.reshape(B, L, D)
